# baseline (speedup 1.0000x reference)
import functools

import jax
import jax.numpy as jnp
from jax import lax
from jax.experimental import pallas as pl
from jax.experimental.pallas import tpu as pltpu
from jax.experimental.pallas import tpu_sc as plsc

_B = 4096
_L = 50
_D = 768
_N = _B * _L
_NE = 7
_NC = 2
_NS = 16
_NW = _NC * _NS
_BPW = _N // _NW
_PW = 2 * _D
_NG = _BPW // 32


def _sc_lookup(idx_flat, table_flat):
    mesh = plsc.VectorSubcoreMesh(core_axis_name="c", subcore_axis_name="s")

    @functools.partial(
        pl.kernel,
        mesh=mesh,
        out_type=jax.ShapeDtypeStruct((_N * _D,), jnp.float32),
        scratch_types=[
            pltpu.VMEM((_NE * _NE * _PW,), jnp.float32),
            pltpu.VMEM((_BPW,), jnp.int32),
            pltpu.SemaphoreType.DMA,
            pltpu.SemaphoreType.DMA,
        ],
        compiler_params=pltpu.CompilerParams(needs_layout_passes=False),
    )
    def k(table_hbm, idx_hbm, out_hbm, pairs_v, idx_v, psem, wsem):
        wid = lax.axis_index("s") * _NC + lax.axis_index("c")
        base = wid * _BPW
        pltpu.sync_copy(idx_hbm.at[pl.ds(base, _BPW)], idx_v)

        for e0 in range(_NE):
            for e1 in range(_NE):
                p = e0 * _NE + e1
                pltpu.async_copy(
                    table_hbm.at[pl.ds(e0 * _D, _D)],
                    pairs_v.at[pl.ds(p * _PW, _D)], psem)
                pltpu.async_copy(
                    table_hbm.at[pl.ds(e1 * _D, _D)],
                    pairs_v.at[pl.ds(p * _PW + _D, _D)], psem)

        def pdrain(j, carry):
            pltpu.make_async_copy(
                table_hbm.at[pl.ds(0, _D)],
                pairs_v.at[pl.ds(0, _D)], psem).wait()
            return carry

        lax.fori_loop(0, 2 * _NE * _NE, pdrain, 0)

        offs2 = lax.iota(jnp.int32, 16) * 2

        def gbody(g, carry):
            e0 = plsc.load_gather(idx_v, [offs2 + (32 * g)])
            e1 = plsc.load_gather(idx_v, [offs2 + (32 * g + 1)])
            pr = e0 * _NE + e1
            r0 = (base + 32 * g) * _D
            for kk in range(16):
                p_off = pl.multiple_of(pr[kk] * _PW, 8)
                pltpu.async_copy(
                    pairs_v.at[pl.ds(p_off, _PW)],
                    out_hbm.at[pl.ds(r0 + kk * _PW, _PW)],
                    wsem)
            return carry

        lax.fori_loop(0, _NG, gbody, 0)

        def drain(j, carry):
            pltpu.make_async_copy(
                pairs_v.at[pl.ds(0, _PW)],
                out_hbm.at[pl.ds(base * _D, _PW)], wsem).wait()
            return carry

        lax.fori_loop(0, _BPW // 2, drain, 0)

    return k(table_flat, idx_flat)


def kernel(detect_emo, emo_emb):
    idx = detect_emo.reshape(_N).astype(jnp.int32)
    out = _sc_lookup(idx, emo_emb.reshape(_NE * _D))
    return out.reshape(_B, _L, _D)


# speedup vs baseline: 1.9451x; 1.9451x over previous
import functools

import jax
import jax.numpy as jnp
from jax import lax
from jax.experimental import pallas as pl
from jax.experimental.pallas import tpu as pltpu
from jax.experimental.pallas import tpu_sc as plsc

_B = 4096
_L = 50
_D = 768
_N = _B * _L
_NE = 7
_NC = 2
_NS = 16
_NW = _NC * _NS
_BW = _B // _NW
_BPW = _N // _NW


def _sc_lookup(idx_flat, table_flat):
    mesh = plsc.VectorSubcoreMesh(core_axis_name="c", subcore_axis_name="s")

    @functools.partial(
        pl.kernel,
        mesh=mesh,
        out_type=jax.ShapeDtypeStruct((_B, _L, _D), jnp.float32),
        scratch_types=[
            pltpu.VMEM((_NE * _D,), jnp.float32),
            pltpu.VMEM((_BPW,), jnp.int32),
            pltpu.SemaphoreType.DMA,
        ],
        compiler_params=pltpu.CompilerParams(
            needs_layout_passes=False, use_tc_tiling_on_sc=True),
    )
    def k(table_hbm, idx_hbm, out_hbm, table_v, idx_v, wsem):
        wid = lax.axis_index("s") * _NC + lax.axis_index("c")
        base = wid * _BPW
        pltpu.sync_copy(table_hbm, table_v)
        pltpu.sync_copy(idx_hbm.at[pl.ds(base, _BPW)], idx_v)

        def bbody(bl, carry):
            b = wid * _BW + bl
            r0 = bl * _L
            ev0 = idx_v[pl.ds(r0, 16)]
            ev1 = idx_v[pl.ds(r0 + 16, 16)]
            ev2 = idx_v[pl.ds(r0 + 32, 16)]
            ev3 = idx_v[pl.ds(r0 + 34, 16)]

            def row(l, e):
                e_off = pl.multiple_of(e * _D, 8)
                pltpu.async_copy(
                    table_v.at[pl.ds(e_off, _D)],
                    out_hbm.at[b, l],
                    wsem)

            for kk in range(16):
                row(kk, ev0[kk])
                row(16 + kk, ev1[kk])
                row(32 + kk, ev2[kk])
            row(48, ev3[14])
            row(49, ev3[15])
            return carry

        lax.fori_loop(0, _BW, bbody, 0)

        def drain(j, carry):
            pltpu.make_async_copy(
                table_v.at[pl.ds(0, _D)], out_hbm.at[0, 0], wsem).wait()
            return carry

        lax.fori_loop(0, _BPW, drain, 0)

    return k(table_flat, idx_flat)


def kernel(detect_emo, emo_emb):
    idx = detect_emo.reshape(_N).astype(jnp.int32)
    return _sc_lookup(idx, emo_emb.reshape(_NE * _D))


# speedup vs baseline: 5.1081x; 2.6261x over previous
import functools

import jax
import jax.numpy as jnp
from jax import lax
from jax.experimental import pallas as pl
from jax.experimental.pallas import tpu as pltpu
from jax.experimental.pallas import tpu_sc as plsc

_B = 4096
_L = 50
_D = 768
_N = _B * _L
_NE = 7
_NC = 2
_NS = 16
_NW = _NC * _NS
_BW = _B // _NW
_BPW = _N // _NW


def _sc_lookup(idx_flat, table_flat):
    mesh = plsc.VectorSubcoreMesh(core_axis_name="c", subcore_axis_name="s")

    @functools.partial(
        pl.kernel,
        mesh=mesh,
        out_type=jax.ShapeDtypeStruct((_L, _B, _D), jnp.float32),
        scratch_types=[
            pltpu.VMEM((_NE * _D,), jnp.float32),
            pltpu.VMEM((_BPW,), jnp.int32),
            pltpu.SemaphoreType.DMA,
        ],
        compiler_params=pltpu.CompilerParams(
            needs_layout_passes=False, use_tc_tiling_on_sc=True),
    )
    def k(table_hbm, idx_hbm, out_hbm, table_v, idx_v, wsem):
        wid = lax.axis_index("s") * _NC + lax.axis_index("c")
        base = wid * _BPW
        pltpu.sync_copy(table_hbm, table_v)
        pltpu.sync_copy(idx_hbm.at[pl.ds(base, _BPW)], idx_v)

        def bbody(bl, carry):
            b = wid * _BW + bl
            r0 = bl * _L
            ev0 = idx_v[pl.ds(r0, 16)]
            ev1 = idx_v[pl.ds(r0 + 16, 16)]
            ev2 = idx_v[pl.ds(r0 + 32, 16)]
            ev3 = idx_v[pl.ds(r0 + 34, 16)]

            def row(l, e):
                e_off = pl.multiple_of(e * _D, 8)
                pltpu.async_copy(
                    table_v.at[pl.ds(e_off, _D)],
                    out_hbm.at[l, b],
                    wsem)

            for kk in range(16):
                row(kk, ev0[kk])
                row(16 + kk, ev1[kk])
                row(32 + kk, ev2[kk])
            row(48, ev3[14])
            row(49, ev3[15])
            return carry

        lax.fori_loop(0, _BW, bbody, 0)

        def drain(j, carry):
            pltpu.make_async_copy(
                table_v.at[pl.ds(0, _D)], out_hbm.at[0, 0], wsem).wait()
            return carry

        lax.fori_loop(0, _BPW, drain, 0)

    return k(table_flat, idx_flat)


def kernel(detect_emo, emo_emb):
    idx = detect_emo.reshape(_N).astype(jnp.int32)
    out_lbd = _sc_lookup(idx, emo_emb.reshape(_NE * _D))
    return jnp.transpose(out_lbd, (1, 0, 2))


# speedup vs baseline: 5.8117x; 1.1377x over previous
import functools

import jax
import jax.numpy as jnp
from jax import lax
from jax.experimental import pallas as pl
from jax.experimental.pallas import tpu as pltpu
from jax.experimental.pallas import tpu_sc as plsc

_B = 4096
_L = 50
_D = 768
_N = _B * _L
_NE = 7
_NC = 2
_NS = 16
_NW = _NC * _NS
_BW = _B // _NW
_BPW = _N // _NW


def _sc_lookup(idx_flat, table_flat):
    mesh = plsc.VectorSubcoreMesh(core_axis_name="c", subcore_axis_name="s")

    @functools.partial(
        pl.kernel,
        mesh=mesh,
        out_type=jax.ShapeDtypeStruct((_L, _B, _D), jnp.float32),
        scratch_types=[
            pltpu.VMEM((_NE * _D,), jnp.float32),
            pltpu.VMEM((_BPW,), jnp.int32),
            pltpu.VMEM((98304,), jnp.int32),
            pltpu.SemaphoreType.DMA,
        ],
        compiler_params=pltpu.CompilerParams(
            needs_layout_passes=False, use_tc_tiling_on_sc=True),
    )
    def k(table_hbm, idx_hbm, out_hbm, table_v, idx_v, dummy_v, wsem):
        wid = lax.axis_index("s") * _NC + lax.axis_index("c")
        base = wid * _BPW
        pltpu.sync_copy(table_hbm, table_v)
        pltpu.sync_copy(idx_hbm.at[pl.ds(base, _BPW)], idx_v)

        def bbody(bl, carry):
            b = wid * _BW + bl
            r0 = bl * _L
            ev0 = idx_v[pl.ds(r0, 16)]
            ev1 = idx_v[pl.ds(r0 + 16, 16)]
            ev2 = idx_v[pl.ds(r0 + 32, 16)]
            ev3 = idx_v[pl.ds(r0 + 34, 16)]

            def row(l, e):
                e_off = pl.multiple_of(e * _D, 8)
                pltpu.async_copy(
                    table_v.at[pl.ds(e_off, _D)],
                    out_hbm.at[l, b],
                    wsem)

            for kk in range(16):
                row(kk, ev0[kk])
                row(16 + kk, ev1[kk])
                row(32 + kk, ev2[kk])
            row(48, ev3[14])
            row(49, ev3[15])
            return carry

        lax.fori_loop(0, _BW, bbody, 0)

        def drain(j, carry):
            pltpu.make_async_copy(
                idx_hbm.at[pl.ds(0, 98304)], dummy_v, wsem).wait()
            return carry

        lax.fori_loop(0, _BPW * 3072 // (98304 * 4), drain, 0)

    return k(table_flat, idx_flat)


def kernel(detect_emo, emo_emb):
    idx = detect_emo.reshape(_N).astype(jnp.int32)
    out_lbd = _sc_lookup(idx, emo_emb.reshape(_NE * _D))
    return jnp.transpose(out_lbd, (1, 0, 2))


# speedup vs baseline: 5.8334x; 1.0037x over previous
import functools

import jax
import jax.numpy as jnp
from jax import lax
from jax.experimental import pallas as pl
from jax.experimental.pallas import tpu as pltpu
from jax.experimental.pallas import tpu_sc as plsc

_B = 4096
_L = 50
_D = 768
_N = _B * _L
_NE = 7
_NC = 2
_NS = 16
_NW = _NC * _NS
_BW = _B // _NW
_BPW = _N // _NW
_NBT = _B // 8
_NU = _L * _NBT
_UPW = _NU // _NW


def _sc_lookup(idx_flat, table_flat):
    mesh = plsc.VectorSubcoreMesh(core_axis_name="c", subcore_axis_name="s")

    @functools.partial(
        pl.kernel,
        mesh=mesh,
        out_type=jax.ShapeDtypeStruct((_L, _B, _D), jnp.float32),
        scratch_types=[
            pltpu.VMEM((_NE * _D,), jnp.float32),
            pltpu.VMEM((3 * (_B + 16),), jnp.int32),
            pltpu.VMEM((98304,), jnp.int32),
            pltpu.SemaphoreType.DMA,
        ],
        compiler_params=pltpu.CompilerParams(
            needs_layout_passes=False, use_tc_tiling_on_sc=True),
    )
    def k(table_hbm, idx_hbm, out_hbm, table_v, idx_v, dummy_v, wsem):
        wid = lax.axis_index("s") * _NC + lax.axis_index("c")
        u0 = wid * _UPW
        l0 = u0 // _NBT
        pltpu.sync_copy(table_hbm, table_v)
        for lr in range(3):
            l_src = jnp.minimum(l0 + lr, _L - 1)
            pltpu.sync_copy(
                idx_hbm.at[pl.ds(pl.multiple_of(l_src * _B, 8), _B)],
                idx_v.at[pl.ds(lr * (_B + 16), _B)])

        def ubody(ul, carry):
            u = u0 + ul
            l = u >> 9
            bt = u & (_NBT - 1)
            lr = l - l0
            ev = idx_v[pl.ds(lr * (_B + 16) + 8 * bt, 16)]

            for kk in range(8):
                e_off = pl.multiple_of(ev[kk] * _D, 8)
                pltpu.async_copy(
                    table_v.at[pl.ds(e_off, _D)],
                    out_hbm.at[l, 8 * bt + kk],
                    wsem)
            return carry

        lax.fori_loop(0, _UPW, ubody, 0)

        def drain(j, carry):
            pltpu.make_async_copy(
                idx_hbm.at[pl.ds(0, 98304)], dummy_v, wsem).wait()
            return carry

        lax.fori_loop(0, _BPW * 3072 // (98304 * 4), drain, 0)

    return k(table_flat, idx_flat)


def kernel(detect_emo, emo_emb):
    idx = detect_emo.astype(jnp.int32).T.reshape(_N)
    out_lbd = _sc_lookup(idx, emo_emb.reshape(_NE * _D))
    return jnp.transpose(out_lbd, (1, 0, 2))
